# baseline (speedup 1.0000x reference)
import functools

import jax
import jax.numpy as jnp
from jax import lax
from jax.experimental import pallas as pl
from jax.experimental.pallas import tpu as pltpu
from jax.experimental.pallas import tpu_sc as plsc

D = 64
L = 16
NC = 2
NS = 16
NW = NC * NS
NU_RAW = 100001
NUP = 100352
NTP = 2 * NUP
NRANGE = 16
RNG = NTP // NRANGE
STRIPE = RNG // NS
ZR = STRIPE // 8
CEDGE = 200
BATCH = 4096
E_G = 800000
E_B = 400000
LAMB = 0.5
BETA = 0.5
REG = 1e-4
R_TC = 128


def _mesh():
    return plsc.VectorSubcoreMesh(core_axis_name="c", subcore_axis_name="s")


@functools.cache
def _deg_kernel():
    DS = NTP // NS
    ZB = DS // 4

    def body(gu, gv, u0, v0, u1, v1, u2, v2, out, ub, vb, ones, zb, degs):
        cid = lax.axis_index("c")
        sid = lax.axis_index("s")
        w = cid * NS + sid

        def zfill(i, _):
            zb[pl.ds(i * L, L)] = jnp.zeros((L,), jnp.float32)
            return ()
        lax.fori_loop(0, ZB // L, zfill, ())

        def ofill(i, _):
            ones[pl.ds(i * L, L)] = jnp.ones((L,), jnp.float32)
            return ()
        lax.fori_loop(0, CEDGE // L, ofill, ())

        for g, (eu, ev, ne) in enumerate(
            ((gu, gv, E_G), (u0, v0, E_B), (u1, v1, E_B), (u2, v2, E_B))):
            def zcopy(j, _):
                pltpu.sync_copy(zb, degs.at[pl.ds(sid * DS + j * ZB, ZB)])
                return ()
            lax.fori_loop(0, DS // ZB, zcopy, ())
            plsc.subcore_barrier()

            def chunk(ci):
                off = ci * CEDGE
                pltpu.sync_copy(eu.at[pl.ds(off, CEDGE)], ub)
                pltpu.sync_copy(ev.at[pl.ds(off, CEDGE)], vb)

                def fix(i, _):
                    s = pl.ds(i * L, L)
                    vb[s] = vb[s] + jnp.full((L,), NUP, jnp.int32)
                    return ()
                lax.fori_loop(0, CEDGE // L, fix, ())
                pltpu.sync_copy(ones, degs.at[ub], add=True)
                pltpu.sync_copy(ones, degs.at[vb], add=True)

            nch = ne // CEDGE
            full, rem = nch // NW, nch % NW

            def cb(j, _):
                chunk(w + j * NW)
                return ()
            lax.fori_loop(0, full, cb, ())
            if rem:
                @pl.when(w < rem)
                def _():
                    chunk(full * NW + w)

            plsc.subcore_barrier()
            pltpu.sync_copy(
                degs.at[pl.ds(sid * DS, DS)],
                out.at[pl.ds(cid * 4 * NTP + g * NTP + sid * DS, DS)])

    return pl.kernel(
        body,
        out_type=jax.ShapeDtypeStruct((NC * 4 * NTP,), jnp.float32),
        mesh=_mesh(),
        compiler_params=pltpu.CompilerParams(use_tc_tiling_on_sc=False),
        scratch_types=[
            pltpu.VMEM((CEDGE,), jnp.int32),
            pltpu.VMEM((CEDGE,), jnp.int32),
            pltpu.VMEM((CEDGE,), jnp.float32),
            pltpu.VMEM((ZB,), jnp.float32),
            pltpu.VMEM_SHARED((NTP,), jnp.float32),
        ],
    )


@functools.cache
def _seg_kernel(mode):
    ncpr_full = NRANGE // NC
    ncpr_user = NRANGE // (2 * NC)
    if mode == "s1":
        out_rows = 4 * NTP
        cfgs = [dict(g=g, both=True, ncpr=ncpr_full, out_base=g * NTP)
                for g in range(4)]
    else:
        out_rows = NTP + 3 * NUP
        cfgs = [dict(g=0, both=True, ncpr=ncpr_full, out_base=0)]
        cfgs += [dict(g=g, both=False, ncpr=ncpr_user,
                      out_base=NTP + (g - 1) * NUP)
                 for g in range(1, 4)]

    def body(f, gu, gv, u0, v0, u1, v1, u2, v2, out,
             ub_0, vb_0, ub_1, vb_1,
             sa_0, da_0, sb_0, db_0, sa_1, da_1, sb_1, db_1,
             ra_0, rb_0, ra_1, rb_1, zb, acc,
             si_0, si_1, sga_0, sgb_0, sga_1, sgb_1,
             ssa_0, ssb_0, ssa_1, ssb_1):
        cid = lax.axis_index("c")
        sid = lax.axis_index("s")
        edges_all = ((gu, gv), (u0, v0), (u1, v1), (u2, v2))
        ne_all = (E_G, E_B, E_B, E_B)
        ubs = (ub_0, ub_1)
        vbs = (vb_0, vb_1)
        sas = (sa_0, sa_1)
        das = (da_0, da_1)
        sbs = (sb_0, sb_1)
        dbs = (db_0, db_1)
        ras = (ra_0, ra_1)
        rbs = (rb_0, rb_1)
        sis = (si_0, si_1)
        sgas = (sga_0, sga_1)
        sgbs = (sgb_0, sgb_1)
        ssas = (ssa_0, ssa_1)
        ssbs = (ssb_0, ssb_1)

        def zfill(i, _):
            zb[i // (D // L), pl.ds((i % (D // L)) * L, L)] = (
                jnp.zeros((L,), jnp.float32))
            return ()
        lax.fori_loop(0, ZR * (D // L), zfill, ())

        for cfg in cfgs:
            g = cfg["g"]
            eu, ev = edges_all[g]
            ne = ne_all[g]
            ncpr = cfg["ncpr"]
            both = cfg["both"]
            out_base = cfg["out_base"]
            fb = g * NTP
            nch = ne // CEDGE
            T, rem = nch // NS, nch % NS
            T2 = T // 2

            def rtask(r_i):
                rbase = (cid * ncpr + r_i) * RNG

                def zcopy(j, _):
                    pltpu.sync_copy(zb,
                                    acc.at[pl.ds(sid * STRIPE + j * ZR, ZR), :])
                    return ()
                lax.fori_loop(0, STRIPE // ZR, zcopy, ())
                plsc.subcore_barrier()

                offA = jnp.full((L,), NUP - rbase, jnp.int32)
                offB = jnp.full((L,), -rbase, jnp.int32)
                fbA = jnp.full((L,), fb, jnp.int32)
                fbB = jnp.full((L,), fb + NUP, jnp.int32)
                neg1 = jnp.full((L,), -1, jnp.int32)
                rngc = jnp.uint32(RNG)

                def issue_idx(t, p):
                    off = (sid + t * NS) * CEDGE
                    pltpu.async_copy(eu.at[pl.ds(off, CEDGE)], ubs[p], sis[p])
                    pltpu.async_copy(ev.at[pl.ds(off, CEDGE)], vbs[p], sis[p])

                def wait_idx(p):
                    pltpu.make_async_copy(
                        eu.at[pl.ds(0, CEDGE)], ubs[p], sis[p]).wait()
                    pltpu.make_async_copy(
                        ev.at[pl.ds(0, CEDGE)], vbs[p], sis[p]).wait()

                def filters(p):
                    def filt(i, _):
                        s = pl.ds(i * L, L)
                        u = ubs[p][s]
                        v = vbs[p][s]
                        if both:
                            dl = v + offA
                            m = lax.bitcast_convert_type(dl, jnp.uint32) < rngc
                            das[p][s] = jnp.where(m, dl, neg1)
                            sas[p][s] = jnp.where(m, u + fbA, neg1)
                        dl2 = u + offB
                        m2 = lax.bitcast_convert_type(dl2, jnp.uint32) < rngc
                        dbs[p][s] = jnp.where(m2, dl2, neg1)
                        sbs[p][s] = jnp.where(m2, v + fbB, neg1)
                        return ()
                    lax.fori_loop(0, CEDGE // L, filt, ())

                def issue_gathers(p):
                    if both:
                        pltpu.async_copy(
                            f.at[plsc.Indices(sas[p], ignored_value=-1)],
                            ras[p], sgas[p])
                    pltpu.async_copy(
                        f.at[plsc.Indices(sbs[p], ignored_value=-1)],
                        rbs[p], sgbs[p])

                def wait_gathers_issue_scatters(p):
                    if both:
                        pltpu.make_async_copy(
                            f.at[plsc.Indices(sas[p], ignored_value=-1)],
                            ras[p], sgas[p]).wait()
                        pltpu.async_copy(
                            ras[p],
                            acc.at[plsc.Indices(das[p], ignored_value=-1)],
                            ssas[p], add=True)
                    pltpu.make_async_copy(
                        f.at[plsc.Indices(sbs[p], ignored_value=-1)],
                        rbs[p], sgbs[p]).wait()
                    pltpu.async_copy(
                        rbs[p],
                        acc.at[plsc.Indices(dbs[p], ignored_value=-1)],
                        ssbs[p], add=True)

                def wait_scatters(p):
                    if both:
                        pltpu.make_async_copy(
                            ras[p],
                            acc.at[plsc.Indices(das[p], ignored_value=-1)],
                            ssas[p]).wait()
                    pltpu.make_async_copy(
                        rbs[p],
                        acc.at[plsc.Indices(dbs[p], ignored_value=-1)],
                        ssbs[p]).wait()

                if T2 > 0:
                    issue_idx(0, 0)

                    def pair(k, _):
                        @pl.when(k > 0)
                        def _():
                            wait_scatters(0)
                        wait_idx(0)
                        filters(0)
                        issue_idx(2 * k + 1, 1)
                        issue_gathers(0)
                        wait_gathers_issue_scatters(0)

                        @pl.when(k > 0)
                        def _():
                            wait_scatters(1)
                        wait_idx(1)
                        filters(1)

                        @pl.when(k < T2 - 1)
                        def _():
                            issue_idx(2 * k + 2, 0)
                        issue_gathers(1)
                        wait_gathers_issue_scatters(1)
                        return ()
                    lax.fori_loop(0, T2, pair, ())
                    wait_scatters(0)
                    wait_scatters(1)

                def sync_chunk(ci):
                    pltpu.sync_copy(eu.at[pl.ds(ci * CEDGE, CEDGE)], ubs[0])
                    pltpu.sync_copy(ev.at[pl.ds(ci * CEDGE, CEDGE)], vbs[0])
                    filters(0)
                    issue_gathers(0)
                    wait_gathers_issue_scatters(0)
                    wait_scatters(0)

                for t_extra in range(2 * T2, T):
                    sync_chunk(sid + t_extra * NS)
                if rem:
                    @pl.when(sid < rem)
                    def _():
                        sync_chunk(T * NS + sid)

                plsc.subcore_barrier()
                orow = pl.multiple_of(out_base + rbase + sid * STRIPE, 8)
                pltpu.sync_copy(acc.at[pl.ds(sid * STRIPE, STRIPE), :],
                                out.at[pl.ds(orow, STRIPE), :])

            def rloop(r, _):
                rtask(r)
                return ()
            lax.fori_loop(0, ncpr, rloop, ())

    return pl.kernel(
        body,
        out_type=jax.ShapeDtypeStruct((out_rows, D), jnp.float32),
        mesh=_mesh(),
        compiler_params=pltpu.CompilerParams(use_tc_tiling_on_sc=False),
        scratch_types=(
            [pltpu.VMEM((CEDGE,), jnp.int32) for _ in range(12)]
            + [pltpu.VMEM((CEDGE, D), jnp.float32) for _ in range(4)]
            + [pltpu.VMEM((ZR, D), jnp.float32),
               pltpu.VMEM_SHARED((RNG, D), jnp.float32)]
            + [pltpu.SemaphoreType.DMA for _ in range(10)]
        ),
    )


@functools.cache
def _gather_kernel():
    B4 = 4 * BATCH
    B7 = 7 * BATCH

    def body(s1f, s2f, e0r, nf, idx_us1, idx_us2, idx_it, sids,
             us1, us2, is1, is2, uw, iw, ns, ni, ib, rb, eb):
        cid = lax.axis_index("c")
        sid = lax.axis_index("s")
        w = cid * NS + sid

        row_tasks = (
            (s1f, idx_us1, us1, B4),
            (s2f, idx_us2, us2, B4),
            (s1f, idx_it, is1, B7),
            (s2f, idx_it, is2, B7),
            (e0r, sids, uw, BATCH),
            (e0r, idx_it, iw, B7),
        )
        for src, idxa, outr, tot in row_tasks:
            n = tot // NW
            pltpu.sync_copy(idxa.at[pl.ds(w * n, n)], ib.at[pl.ds(0, n)])
            done = 0
            while done < n:
                sn = min(512, n - done)
                pltpu.sync_copy(src.at[ib.at[pl.ds(done, sn)]],
                                rb.at[pl.ds(0, sn), :])
                pltpu.sync_copy(rb.at[pl.ds(0, sn), :],
                                outr.at[pl.ds(w * n + done, sn), :])
                done += sn

        for idxa, outr, tot in ((idx_us1, ns, B4), (idx_it, ni, B7)):
            n = tot // NW
            pltpu.sync_copy(idxa.at[pl.ds(w * n, n)], ib.at[pl.ds(0, n)])
            pltpu.sync_copy(nf.at[ib.at[pl.ds(0, n)]], eb.at[pl.ds(0, n)])
            pltpu.sync_copy(eb.at[pl.ds(0, n)], outr.at[pl.ds(w * n, n)])

    return pl.kernel(
        body,
        out_type=(
            jax.ShapeDtypeStruct((B4, D), jnp.float32),
            jax.ShapeDtypeStruct((B4, D), jnp.float32),
            jax.ShapeDtypeStruct((B7, D), jnp.float32),
            jax.ShapeDtypeStruct((B7, D), jnp.float32),
            jax.ShapeDtypeStruct((BATCH, D), jnp.float32),
            jax.ShapeDtypeStruct((B7, D), jnp.float32),
            jax.ShapeDtypeStruct((B4,), jnp.float32),
            jax.ShapeDtypeStruct((B7,), jnp.float32),
        ),
        mesh=_mesh(),
        compiler_params=pltpu.CompilerParams(use_tc_tiling_on_sc=False),
        scratch_types=[
            pltpu.VMEM((B7 // NW,), jnp.int32),
            pltpu.VMEM((512, D), jnp.float32),
            pltpu.VMEM((B7 // NW,), jnp.float32),
        ],
    )


def _tc_ssq(uwp, iwp):
    grid = NUP // R_TC

    def body(u_ref, i_ref, ssq_ref):
        ub = u_ref[...]
        ib = i_ref[...]
        su = jnp.sum(ub * ub)
        si = jnp.sum(ib * ib)
        m0 = ((lax.broadcasted_iota(jnp.int32, (2, 128), 0) == 0)
              & (lax.broadcasted_iota(jnp.int32, (2, 128), 1) == 0))
        m1 = ((lax.broadcasted_iota(jnp.int32, (2, 128), 0) == 1)
              & (lax.broadcasted_iota(jnp.int32, (2, 128), 1) == 0))

        @pl.when(pl.program_id(0) == 0)
        def _():
            ssq_ref[...] = jnp.zeros_like(ssq_ref)
        ssq_ref[...] += jnp.where(m0, su, 0.0) + jnp.where(m1, si, 0.0)

    return pl.pallas_call(
        body,
        grid=(grid,),
        in_specs=[
            pl.BlockSpec((R_TC, D), lambda i: (i, 0)),
            pl.BlockSpec((R_TC, D), lambda i: (i, 0)),
        ],
        out_specs=pl.BlockSpec((2, 128), lambda i: (0, 0)),
        out_shape=jax.ShapeDtypeStruct((2, 128), jnp.float32),
    )(uwp, iwp)


RB = 512
NB = BATCH // RB


def _tc_final(us1, us2, uw, ns, ip1, ip2, ipw, npn, in1, in2, inw, nnn,
              ir1, ir2, irw, nrn, ssq, W1, b1, W2, b2):
    def body(us1_r, us2_r, uw_r, ns_r, ip1_r, ip2_r, ipw_r, npn_r,
             in1_r, in2_r, inw_r, nnn_r, ir1_r, ir2_r, irw_r, nrn_r,
             ssq_r, w1_r, b1_r, w2_r, b2_r, out_r, acc_r):
        i = pl.program_id(0)

        ue = (uw_r[...][None] + ns_r[...][:, :, None]
              * (us1_r[...] + us2_r[...])) / 3.0
        pos = (ipw_r[...] + npn_r[...][:, None]
               * (ip1_r[...] + ip2_r[...])) / 3.0
        neg = ((inw_r[...] + nnn_r[...][:, None]
                * (in1_r[...] + in2_r[...])) / 3.0).reshape(RB, 4, D)
        rec = ((irw_r[...] + nrn_r[...][:, None]
                * (ir1_r[...] + ir2_r[...])) / 3.0).reshape(RB, 2, D)

        h = jnp.tanh(jax.lax.dot_general(
            ue.reshape(4 * RB, D), w1_r[...], (((1,), (1,)), ((), ())),
            preferred_element_type=jnp.float32) + b1_r[...])
        inv = jnp.tanh(jax.lax.dot_general(
            h, w2_r[...], (((1,), (1,)), ((), ())),
            preferred_element_type=jnp.float32) + b2_r[...])
        inv = inv.reshape(4, RB, D)

        p_sc = jnp.sum(inv * pos[None], axis=-1)
        n_sc = jnp.einsum("kbd,bjd->kbj", inv, neg,
                          preferred_element_type=jnp.float32)
        pr_p = jnp.clip(jax.nn.sigmoid(p_sc), 1e-7, 1.0 - 1e-7)
        pr_n = jnp.clip(jax.nn.sigmoid(n_sc), 1e-7, 1.0 - 1e-7)
        s_logp = jnp.sum(jnp.log(pr_p)) + jnp.sum(jnp.log(1.0 - pr_n))

        tar = ue[3]
        var = tar - inv[3]
        invm = jnp.mean(inv, axis=0)
        inv_s = jnp.einsum("bd,bjd->bj", invm, rec,
                           preferred_element_type=jnp.float32)
        tar_s = jnp.einsum("bd,bjd->bj", var, rec,
                           preferred_element_type=jnp.float32)
        sc = BETA * inv_s + (1.0 - BETA) * tar_s
        dsc = sc[:, 0] - sc[:, 1]
        s_bpr = jnp.sum(jnp.log(jax.nn.sigmoid(dsc) + 1e-10))

        iota0 = lax.broadcasted_iota(jnp.int32, (8, 128), 0)
        iota1 = lax.broadcasted_iota(jnp.int32, (8, 128), 1)
        part = (jnp.where((iota0 == 0) & (iota1 == 0), s_logp, 0.0)
                + jnp.where((iota0 == 1) & (iota1 == 0), s_bpr, 0.0))

        @pl.when(i == 0)
        def _():
            acc_r[...] = jnp.zeros_like(acc_r)
        acc_r[...] += part

        @pl.when(i == NB - 1)
        def _():
            a = acc_r[...]
            log_loss = -a[0, 0] / (20 * BATCH)
            bpr = -a[1, 0] / BATCH
            ssqv = ssq_r[...]
            reg = REG * (jnp.sqrt(jnp.sum(ssqv[0]))
                         + jnp.sqrt(jnp.sum(ssqv[1]))) / NU_RAW
            out_r[...] = jnp.reshape(
                LAMB * log_loss + (1.0 - LAMB) * bpr + reg, (1, 1))

    def bs(shape, fn):
        return pl.BlockSpec(shape, fn)

    return pl.pallas_call(
        body,
        grid=(NB,),
        in_specs=[
            bs((4, RB, D), lambda i: (0, i, 0)),
            bs((4, RB, D), lambda i: (0, i, 0)),
            bs((RB, D), lambda i: (i, 0)),
            bs((4, RB), lambda i: (0, i)),
            bs((RB, D), lambda i: (i, 0)),
            bs((RB, D), lambda i: (i, 0)),
            bs((RB, D), lambda i: (i, 0)),
            bs((RB,), lambda i: (i,)),
            bs((4 * RB, D), lambda i: (i, 0)),
            bs((4 * RB, D), lambda i: (i, 0)),
            bs((4 * RB, D), lambda i: (i, 0)),
            bs((4 * RB,), lambda i: (i,)),
            bs((2 * RB, D), lambda i: (i, 0)),
            bs((2 * RB, D), lambda i: (i, 0)),
            bs((2 * RB, D), lambda i: (i, 0)),
            bs((2 * RB,), lambda i: (i,)),
            bs((2, 128), lambda i: (0, 0)),
            bs((D, D), lambda i: (0, 0)),
            bs((1, D), lambda i: (0, 0)),
            bs((D, D), lambda i: (0, 0)),
            bs((1, D), lambda i: (0, 0)),
        ],
        out_specs=pl.BlockSpec((1, 1), lambda i: (0, 0)),
        out_shape=jax.ShapeDtypeStruct((1, 1), jnp.float32),
        scratch_shapes=[pltpu.VMEM((8, 128), jnp.float32)],
    )(us1, us2, uw, ns, ip1, ip2, ipw, npn, in1, in2, inw, nnn,
      ir1, ir2, irw, nrn, ssq, W1, b1, W2, b2)


def kernel(batch_data, g_edges, be0, be1, be2, user_w, item_w, W1, b1, W2, b2):
    uwp = jnp.pad(user_w, ((0, NUP - NU_RAW), (0, 0)))
    iwp = jnp.pad(item_w, ((0, NUP - NU_RAW), (0, 0)))
    e0 = jnp.concatenate([uwp, iwp], axis=0)

    sids = batch_data[:, 0]
    pos = batch_data[:, 1]
    neg = batch_data[:, 2:6].reshape(-1)
    rec = batch_data[:, 6:8].reshape(-1)
    iids = jnp.concatenate([pos, neg, rec])
    idx_it = iids + NUP
    ks = jnp.arange(4, dtype=jnp.int32) * NTP
    idx_us1 = (sids[None, :] + ks[:, None]).reshape(-1)
    s2b = jnp.array([0, NTP, NTP + NUP, NTP + 2 * NUP], jnp.int32)
    idx_us2 = (sids[None, :] + s2b[:, None]).reshape(-1)

    earrs = []
    for e in (g_edges, be0, be1, be2):
        earrs.append(e[0])
        earrs.append(e[1])

    degp = _deg_kernel()(*earrs)
    deg = degp[:4 * NTP] + degp[4 * NTP:]
    nrm = lax.rsqrt(jnp.maximum(deg, 1.0))
    f0 = (nrm.reshape(4, NTP, 1) * e0[None]).reshape(4 * NTP, D)
    s1f = _seg_kernel("s1")(f0, *earrs)
    f1 = s1f * (nrm * nrm)[:, None]
    s2f = _seg_kernel("s2")(f1, *earrs)

    us1, us2, is1, is2, uw, iw, ns, ni = _gather_kernel()(
        s1f, s2f, e0, nrm, idx_us1, idx_us2, idx_it, sids)

    ssq = _tc_ssq(uwp, iwp)
    BB = BATCH
    sp = [BB, 5 * BB]
    ip1, in1, ir1 = jnp.split(is1, sp)
    ip2, in2, ir2 = jnp.split(is2, sp)
    ipw, inw, irw = jnp.split(iw, sp)
    npn, nnn, nrn = jnp.split(ni, sp)
    out = _tc_final(us1.reshape(4, BATCH, D), us2.reshape(4, BATCH, D), uw,
                    ns.reshape(4, BATCH), ip1, ip2, ipw, npn,
                    in1, in2, inw, nnn, ir1, ir2, irw, nrn, ssq,
                    W1, b1.reshape(1, D), W2, b2.reshape(1, D))
    return out.reshape(())


# speedup vs baseline: 4.1615x; 4.1615x over previous
import functools

import jax
import jax.numpy as jnp
from jax import lax
from jax.experimental import pallas as pl
from jax.experimental.pallas import tpu as pltpu
from jax.experimental.pallas import tpu_sc as plsc

D = 64
L = 16
NC = 2
NS = 16
NW = NC * NS
NU_RAW = 100001
NUP = 100352
NTP = 2 * NUP
NRANGE = 16
RNG = NTP // NRANGE
STRIPE = RNG // NS
ZR = STRIPE // 8
CEDGE = 400
BATCH = 4096
E_G = 800000
E_B = 400000
LAMB = 0.5
BETA = 0.5
REG = 1e-4
R_TC = 128


def _mesh():
    return plsc.VectorSubcoreMesh(core_axis_name="c", subcore_axis_name="s")


@functools.cache
def _deg_kernel():
    DS = NTP // NS
    ZB = DS // 4

    def body(gu, gv, u0, v0, u1, v1, u2, v2, out, ub, vb, ones, zb, degs):
        cid = lax.axis_index("c")
        sid = lax.axis_index("s")
        w = cid * NS + sid

        def zfill(i, _):
            zb[pl.ds(i * L, L)] = jnp.zeros((L,), jnp.float32)
            return ()
        lax.fori_loop(0, ZB // L, zfill, ())

        def ofill(i, _):
            ones[pl.ds(i * L, L)] = jnp.ones((L,), jnp.float32)
            return ()
        lax.fori_loop(0, CEDGE // L, ofill, ())

        for g, (eu, ev, ne) in enumerate(
            ((gu, gv, E_G), (u0, v0, E_B), (u1, v1, E_B), (u2, v2, E_B))):
            def zcopy(j, _):
                pltpu.sync_copy(zb, degs.at[pl.ds(sid * DS + j * ZB, ZB)])
                return ()
            lax.fori_loop(0, DS // ZB, zcopy, ())
            plsc.subcore_barrier()

            def chunk(ci):
                off = ci * CEDGE
                pltpu.sync_copy(eu.at[pl.ds(off, CEDGE)], ub)
                pltpu.sync_copy(ev.at[pl.ds(off, CEDGE)], vb)

                def fix(i, _):
                    s = pl.ds(i * L, L)
                    vb[s] = vb[s] + jnp.full((L,), NUP, jnp.int32)
                    return ()
                lax.fori_loop(0, CEDGE // L, fix, ())
                pltpu.sync_copy(ones, degs.at[ub], add=True)
                pltpu.sync_copy(ones, degs.at[vb], add=True)

            nch = ne // CEDGE
            full, rem = nch // NW, nch % NW

            def cb(j, _):
                chunk(w + j * NW)
                return ()
            lax.fori_loop(0, full, cb, ())
            if rem:
                @pl.when(w < rem)
                def _():
                    chunk(full * NW + w)

            plsc.subcore_barrier()
            pltpu.sync_copy(
                degs.at[pl.ds(sid * DS, DS)],
                out.at[pl.ds(cid * 4 * NTP + g * NTP + sid * DS, DS)])

    return pl.kernel(
        body,
        out_type=jax.ShapeDtypeStruct((NC * 4 * NTP,), jnp.float32),
        mesh=_mesh(),
        compiler_params=pltpu.CompilerParams(use_tc_tiling_on_sc=False),
        scratch_types=[
            pltpu.VMEM((CEDGE,), jnp.int32),
            pltpu.VMEM((CEDGE,), jnp.int32),
            pltpu.VMEM((CEDGE,), jnp.float32),
            pltpu.VMEM((ZB,), jnp.float32),
            pltpu.VMEM_SHARED((NTP,), jnp.float32),
        ],
    )


@functools.cache
def _seg_kernel(mode):
    ncpr_full = NRANGE // NC
    ncpr_user = NRANGE // (2 * NC)
    if mode == "s1":
        out_rows = 4 * NTP
        cfgs = [dict(g=g, both=True, ncpr=ncpr_full, out_base=g * NTP)
                for g in range(4)]
    else:
        out_rows = NTP + 3 * NUP
        cfgs = [dict(g=0, both=True, ncpr=ncpr_full, out_base=0)]
        cfgs += [dict(g=g, both=False, ncpr=ncpr_user,
                      out_base=NTP + (g - 1) * NUP)
                 for g in range(1, 4)]

    def body(f, gu, gv, u0, v0, u1, v1, u2, v2, out,
             ub, vb, sa, da, sb, db, ra, rb, zb, acc,
             sga, sgb, ssa, ssb):
        cid = lax.axis_index("c")
        sid = lax.axis_index("s")
        edges_all = ((gu, gv), (u0, v0), (u1, v1), (u2, v2))
        ne_all = (E_G, E_B, E_B, E_B)

        def zfill(i, _):
            zb[i // (D // L), pl.ds((i % (D // L)) * L, L)] = (
                jnp.zeros((L,), jnp.float32))
            return ()
        lax.fori_loop(0, ZR * (D // L), zfill, ())

        for cfg in cfgs:
            g = cfg["g"]
            eu, ev = edges_all[g]
            ne = ne_all[g]
            ncpr = cfg["ncpr"]
            both = cfg["both"]
            out_base = cfg["out_base"]
            fb = g * NTP
            nch = ne // CEDGE
            T, rem = nch // NS, nch % NS

            def rtask(r_i):
                rbase = (cid * ncpr + r_i) * RNG

                def zcopy(j, _):
                    pltpu.sync_copy(zb,
                                    acc.at[pl.ds(sid * STRIPE + j * ZR, ZR), :])
                    return ()
                lax.fori_loop(0, STRIPE // ZR, zcopy, ())
                plsc.subcore_barrier()

                offA = jnp.full((L,), NUP - rbase, jnp.int32)
                offB = jnp.full((L,), -rbase, jnp.int32)
                fbA = jnp.full((L,), fb, jnp.int32)
                fbB = jnp.full((L,), fb + NUP, jnp.int32)
                neg1 = jnp.full((L,), -1, jnp.int32)
                rngc = jnp.uint32(RNG)

                def chunk(ci):
                    off = ci * CEDGE
                    pltpu.async_copy(eu.at[pl.ds(off, CEDGE)], ub, sga)
                    pltpu.async_copy(ev.at[pl.ds(off, CEDGE)], vb, sgb)
                    pltpu.make_async_copy(
                        eu.at[pl.ds(0, CEDGE)], ub, sga).wait()
                    pltpu.make_async_copy(
                        ev.at[pl.ds(0, CEDGE)], vb, sgb).wait()

                    def filt(i, _):
                        s = pl.ds(i * L, L)
                        u = ub[s]
                        v = vb[s]
                        if both:
                            dl = v + offA
                            m = lax.bitcast_convert_type(dl, jnp.uint32) < rngc
                            da[s] = jnp.where(m, dl, neg1)
                            sa[s] = jnp.where(m, u + fbA, neg1)
                        dl2 = u + offB
                        m2 = lax.bitcast_convert_type(dl2, jnp.uint32) < rngc
                        db[s] = jnp.where(m2, dl2, neg1)
                        sb[s] = jnp.where(m2, v + fbB, neg1)
                        return ()
                    lax.fori_loop(0, CEDGE // L, filt, ())

                    if both:
                        pltpu.async_copy(
                            f.at[plsc.Indices(sa, ignored_value=-1)], ra, sga)
                    pltpu.async_copy(
                        f.at[plsc.Indices(sb, ignored_value=-1)], rb, sgb)
                    if both:
                        pltpu.make_async_copy(
                            f.at[plsc.Indices(sa, ignored_value=-1)],
                            ra, sga).wait()
                        pltpu.async_copy(
                            ra, acc.at[plsc.Indices(da, ignored_value=-1)],
                            ssa, add=True)
                    pltpu.make_async_copy(
                        f.at[plsc.Indices(sb, ignored_value=-1)],
                        rb, sgb).wait()
                    pltpu.async_copy(
                        rb, acc.at[plsc.Indices(db, ignored_value=-1)],
                        ssb, add=True)
                    if both:
                        pltpu.make_async_copy(
                            ra, acc.at[plsc.Indices(da, ignored_value=-1)],
                            ssa).wait()
                    pltpu.make_async_copy(
                        rb, acc.at[plsc.Indices(db, ignored_value=-1)],
                        ssb).wait()

                def cb(j, _):
                    chunk(sid + j * NS)
                    return ()
                lax.fori_loop(0, T, cb, ())
                if rem:
                    @pl.when(sid < rem)
                    def _():
                        chunk(T * NS + sid)

                plsc.subcore_barrier()
                orow = pl.multiple_of(out_base + rbase + sid * STRIPE, 8)
                pltpu.sync_copy(acc.at[pl.ds(sid * STRIPE, STRIPE), :],
                                out.at[pl.ds(orow, STRIPE), :])

            def rloop(r, _):
                rtask(r)
                return ()
            lax.fori_loop(0, ncpr, rloop, ())

    return pl.kernel(
        body,
        out_type=jax.ShapeDtypeStruct((out_rows, D), jnp.float32),
        mesh=_mesh(),
        compiler_params=pltpu.CompilerParams(use_tc_tiling_on_sc=False),
        scratch_types=(
            [pltpu.VMEM((CEDGE,), jnp.int32) for _ in range(6)]
            + [pltpu.VMEM((CEDGE, D), jnp.float32) for _ in range(2)]
            + [pltpu.VMEM((ZR, D), jnp.float32),
               pltpu.VMEM_SHARED((RNG, D), jnp.float32)]
            + [pltpu.SemaphoreType.DMA for _ in range(4)]
        ),
    )


@functools.cache
def _gather_kernel():
    B4 = 4 * BATCH
    B7 = 7 * BATCH

    def body(s1f, s2f, e0r, nf, idx_us1, idx_us2, idx_it, sids,
             us1, us2, is1, is2, uw, iw, ns, ni, ib, rb, eb):
        cid = lax.axis_index("c")
        sid = lax.axis_index("s")
        w = cid * NS + sid

        row_tasks = (
            (s1f, idx_us1, us1, B4),
            (s2f, idx_us2, us2, B4),
            (s1f, idx_it, is1, B7),
            (s2f, idx_it, is2, B7),
            (e0r, sids, uw, BATCH),
            (e0r, idx_it, iw, B7),
        )
        for src, idxa, outr, tot in row_tasks:
            n = tot // NW
            pltpu.sync_copy(idxa.at[pl.ds(w * n, n)], ib.at[pl.ds(0, n)])
            done = 0
            while done < n:
                sn = min(512, n - done)
                pltpu.sync_copy(src.at[ib.at[pl.ds(done, sn)]],
                                rb.at[pl.ds(0, sn), :])
                pltpu.sync_copy(rb.at[pl.ds(0, sn), :],
                                outr.at[pl.ds(w * n + done, sn), :])
                done += sn

        for idxa, outr, tot in ((idx_us1, ns, B4), (idx_it, ni, B7)):
            n = tot // NW
            pltpu.sync_copy(idxa.at[pl.ds(w * n, n)], ib.at[pl.ds(0, n)])
            pltpu.sync_copy(nf.at[ib.at[pl.ds(0, n)]], eb.at[pl.ds(0, n)])
            pltpu.sync_copy(eb.at[pl.ds(0, n)], outr.at[pl.ds(w * n, n)])

    return pl.kernel(
        body,
        out_type=(
            jax.ShapeDtypeStruct((B4, D), jnp.float32),
            jax.ShapeDtypeStruct((B4, D), jnp.float32),
            jax.ShapeDtypeStruct((B7, D), jnp.float32),
            jax.ShapeDtypeStruct((B7, D), jnp.float32),
            jax.ShapeDtypeStruct((BATCH, D), jnp.float32),
            jax.ShapeDtypeStruct((B7, D), jnp.float32),
            jax.ShapeDtypeStruct((B4,), jnp.float32),
            jax.ShapeDtypeStruct((B7,), jnp.float32),
        ),
        mesh=_mesh(),
        compiler_params=pltpu.CompilerParams(use_tc_tiling_on_sc=False),
        scratch_types=[
            pltpu.VMEM((B7 // NW,), jnp.int32),
            pltpu.VMEM((512, D), jnp.float32),
            pltpu.VMEM((B7 // NW,), jnp.float32),
        ],
    )


def _tc_ssq(uwp, iwp):
    grid = NUP // R_TC

    def body(u_ref, i_ref, ssq_ref):
        ub = u_ref[...]
        ib = i_ref[...]
        su = jnp.sum(ub * ub)
        si = jnp.sum(ib * ib)
        m0 = ((lax.broadcasted_iota(jnp.int32, (2, 128), 0) == 0)
              & (lax.broadcasted_iota(jnp.int32, (2, 128), 1) == 0))
        m1 = ((lax.broadcasted_iota(jnp.int32, (2, 128), 0) == 1)
              & (lax.broadcasted_iota(jnp.int32, (2, 128), 1) == 0))

        @pl.when(pl.program_id(0) == 0)
        def _():
            ssq_ref[...] = jnp.zeros_like(ssq_ref)
        ssq_ref[...] += jnp.where(m0, su, 0.0) + jnp.where(m1, si, 0.0)

    return pl.pallas_call(
        body,
        grid=(grid,),
        in_specs=[
            pl.BlockSpec((R_TC, D), lambda i: (i, 0)),
            pl.BlockSpec((R_TC, D), lambda i: (i, 0)),
        ],
        out_specs=pl.BlockSpec((2, 128), lambda i: (0, 0)),
        out_shape=jax.ShapeDtypeStruct((2, 128), jnp.float32),
    )(uwp, iwp)


RB = 512
NB = BATCH // RB


def _tc_final(us1, us2, uw, ns, ip1, ip2, ipw, npn, in1, in2, inw, nnn,
              ir1, ir2, irw, nrn, ssq, W1, b1, W2, b2):
    def body(us1_r, us2_r, uw_r, ns_r, ip1_r, ip2_r, ipw_r, npn_r,
             in1_r, in2_r, inw_r, nnn_r, ir1_r, ir2_r, irw_r, nrn_r,
             ssq_r, w1_r, b1_r, w2_r, b2_r, out_r, acc_r):
        i = pl.program_id(0)

        ue = (uw_r[...][None] + ns_r[...][:, :, None]
              * (us1_r[...] + us2_r[...])) / 3.0
        pos = (ipw_r[...] + npn_r[...][:, None]
               * (ip1_r[...] + ip2_r[...])) / 3.0
        neg = ((inw_r[...] + nnn_r[...][:, None]
                * (in1_r[...] + in2_r[...])) / 3.0).reshape(RB, 4, D)
        rec = ((irw_r[...] + nrn_r[...][:, None]
                * (ir1_r[...] + ir2_r[...])) / 3.0).reshape(RB, 2, D)

        h = jnp.tanh(jax.lax.dot_general(
            ue.reshape(4 * RB, D), w1_r[...], (((1,), (1,)), ((), ())),
            preferred_element_type=jnp.float32) + b1_r[...])
        inv = jnp.tanh(jax.lax.dot_general(
            h, w2_r[...], (((1,), (1,)), ((), ())),
            preferred_element_type=jnp.float32) + b2_r[...])
        inv = inv.reshape(4, RB, D)

        p_sc = jnp.sum(inv * pos[None], axis=-1)
        n_sc = jnp.einsum("kbd,bjd->kbj", inv, neg,
                          preferred_element_type=jnp.float32)
        pr_p = jnp.clip(jax.nn.sigmoid(p_sc), 1e-7, 1.0 - 1e-7)
        pr_n = jnp.clip(jax.nn.sigmoid(n_sc), 1e-7, 1.0 - 1e-7)
        s_logp = jnp.sum(jnp.log(pr_p)) + jnp.sum(jnp.log(1.0 - pr_n))

        tar = ue[3]
        var = tar - inv[3]
        invm = jnp.mean(inv, axis=0)
        inv_s = jnp.einsum("bd,bjd->bj", invm, rec,
                           preferred_element_type=jnp.float32)
        tar_s = jnp.einsum("bd,bjd->bj", var, rec,
                           preferred_element_type=jnp.float32)
        sc = BETA * inv_s + (1.0 - BETA) * tar_s
        dsc = sc[:, 0] - sc[:, 1]
        s_bpr = jnp.sum(jnp.log(jax.nn.sigmoid(dsc) + 1e-10))

        iota0 = lax.broadcasted_iota(jnp.int32, (8, 128), 0)
        iota1 = lax.broadcasted_iota(jnp.int32, (8, 128), 1)
        part = (jnp.where((iota0 == 0) & (iota1 == 0), s_logp, 0.0)
                + jnp.where((iota0 == 1) & (iota1 == 0), s_bpr, 0.0))

        @pl.when(i == 0)
        def _():
            acc_r[...] = jnp.zeros_like(acc_r)
        acc_r[...] += part

        @pl.when(i == NB - 1)
        def _():
            a = acc_r[...]
            log_loss = -a[0, 0] / (20 * BATCH)
            bpr = -a[1, 0] / BATCH
            ssqv = ssq_r[...]
            reg = REG * (jnp.sqrt(jnp.sum(ssqv[0]))
                         + jnp.sqrt(jnp.sum(ssqv[1]))) / NU_RAW
            out_r[...] = jnp.reshape(
                LAMB * log_loss + (1.0 - LAMB) * bpr + reg, (1, 1))

    def bs(shape, fn):
        return pl.BlockSpec(shape, fn)

    return pl.pallas_call(
        body,
        grid=(NB,),
        in_specs=[
            bs((4, RB, D), lambda i: (0, i, 0)),
            bs((4, RB, D), lambda i: (0, i, 0)),
            bs((RB, D), lambda i: (i, 0)),
            bs((4, RB), lambda i: (0, i)),
            bs((RB, D), lambda i: (i, 0)),
            bs((RB, D), lambda i: (i, 0)),
            bs((RB, D), lambda i: (i, 0)),
            bs((RB,), lambda i: (i,)),
            bs((4 * RB, D), lambda i: (i, 0)),
            bs((4 * RB, D), lambda i: (i, 0)),
            bs((4 * RB, D), lambda i: (i, 0)),
            bs((4 * RB,), lambda i: (i,)),
            bs((2 * RB, D), lambda i: (i, 0)),
            bs((2 * RB, D), lambda i: (i, 0)),
            bs((2 * RB, D), lambda i: (i, 0)),
            bs((2 * RB,), lambda i: (i,)),
            bs((2, 128), lambda i: (0, 0)),
            bs((D, D), lambda i: (0, 0)),
            bs((1, D), lambda i: (0, 0)),
            bs((D, D), lambda i: (0, 0)),
            bs((1, D), lambda i: (0, 0)),
        ],
        out_specs=pl.BlockSpec((1, 1), lambda i: (0, 0)),
        out_shape=jax.ShapeDtypeStruct((1, 1), jnp.float32),
        scratch_shapes=[pltpu.VMEM((8, 128), jnp.float32)],
    )(us1, us2, uw, ns, ip1, ip2, ipw, npn, in1, in2, inw, nnn,
      ir1, ir2, irw, nrn, ssq, W1, b1, W2, b2)


def kernel(batch_data, g_edges, be0, be1, be2, user_w, item_w, W1, b1, W2, b2):
    uwp = jnp.pad(user_w, ((0, NUP - NU_RAW), (0, 0)))
    iwp = jnp.pad(item_w, ((0, NUP - NU_RAW), (0, 0)))
    e0 = jnp.concatenate([uwp, iwp], axis=0)

    sids = batch_data[:, 0]
    pos = batch_data[:, 1]
    neg = batch_data[:, 2:6].reshape(-1)
    rec = batch_data[:, 6:8].reshape(-1)
    iids = jnp.concatenate([pos, neg, rec])
    idx_it = iids + NUP
    ks = jnp.arange(4, dtype=jnp.int32) * NTP
    idx_us1 = (sids[None, :] + ks[:, None]).reshape(-1)
    s2b = jnp.array([0, NTP, NTP + NUP, NTP + 2 * NUP], jnp.int32)
    idx_us2 = (sids[None, :] + s2b[:, None]).reshape(-1)

    earrs = []
    for e in (g_edges, be0, be1, be2):
        earrs.append(e[0])
        earrs.append(e[1])

    degp = _deg_kernel()(*earrs)
    deg = degp[:4 * NTP] + degp[4 * NTP:]
    nrm = lax.rsqrt(jnp.maximum(deg, 1.0))
    f0 = (nrm.reshape(4, NTP, 1) * e0[None]).reshape(4 * NTP, D)
    s1f = _seg_kernel("s1")(f0, *earrs)
    f1 = s1f * (nrm * nrm)[:, None]
    s2f = _seg_kernel("s2")(f1, *earrs)

    us1, us2, is1, is2, uw, iw, ns, ni = _gather_kernel()(
        s1f, s2f, e0, nrm, idx_us1, idx_us2, idx_it, sids)

    ssq = _tc_ssq(uwp, iwp)
    BB = BATCH
    sp = [BB, 5 * BB]
    ip1, in1, ir1 = jnp.split(is1, sp)
    ip2, in2, ir2 = jnp.split(is2, sp)
    ipw, inw, irw = jnp.split(iw, sp)
    npn, nnn, nrn = jnp.split(ni, sp)
    out = _tc_final(us1.reshape(4, BATCH, D), us2.reshape(4, BATCH, D), uw,
                    ns.reshape(4, BATCH), ip1, ip2, ipw, npn,
                    in1, in2, inw, nnn, ir1, ir2, irw, nrn, ssq,
                    W1, b1.reshape(1, D), W2, b2.reshape(1, D))
    return out.reshape(())
